# baseline (speedup 1.0000x reference)
import functools

import jax
import jax.numpy as jnp
from jax import lax
from jax.experimental import pallas as pl
from jax.experimental.pallas import tpu as pltpu
from jax.experimental.pallas import tpu_sc as plsc

LATENT = 32
BATCH = 16384
NW = 32
B_PER_W = BATCH // NW
CHUNK = 128
NCHUNK = B_PER_W // CHUNK


def _fire_chunk(wi_hbm, ws_hbm, inv_idx, stk_idx, inv_bf, stk_rows, j,
                slot, sem):
    return [
        pltpu.async_copy(wi_hbm.at[inv_idx.at[j]], inv_bf.at[slot], sem),
        pltpu.async_copy(ws_hbm.at[stk_idx.at[j]], stk_rows.at[slot], sem),
    ]


def _sc_kernel(inv_hbm, stk_hbm, wi_hbm, ws_hbm, out_hbm,
               inv_idx, stk_idx, inv_bf, inv_rows, stk_rows, out_v, sems):
    wid = lax.axis_index("s") * 2 + lax.axis_index("c")
    base = wid * NCHUNK

    pltpu.sync_copy(inv_hbm.at[pl.ds(base, NCHUNK)], inv_idx)
    pltpu.sync_copy(stk_hbm.at[pl.ds(base, NCHUNK)], stk_idx)

    pending = _fire_chunk(wi_hbm, ws_hbm, inv_idx, stk_idx,
                          inv_bf, stk_rows, 0, 0, sems.at[0])
    lanes = jnp.arange(16, dtype=jnp.int32)
    for j in range(NCHUNK):
        slot = j % 2
        if j + 1 < NCHUNK:
            nxt = _fire_chunk(wi_hbm, ws_hbm, inv_idx, stk_idx,
                              inv_bf, stk_rows, j + 1, (j + 1) % 2,
                              sems.at[(j + 1) % 2])
        else:
            nxt = []
        for c in pending:
            c.wait()
        pending = nxt

        def unpack_row(r, _):
            row = inv_bf[slot, r, :]
            a, b = plsc.unpack(row, format=plsc.PackFormat.INTERLEAVED)
            inv_rows[slot, r, pl.ds(0, 16)] = a
            inv_rows[slot, r, pl.ds(16, 16)] = b
            return 0
        lax.fori_loop(0, CHUNK, unpack_row, 0)

        def strip(t, _):
            rows = t * 16 + lanes
            acc = jnp.zeros((16,), jnp.float32)
            for d in range(LATENT):
                pos = d // 2 if d % 2 == 0 else 16 + d // 2
                a = plsc.load_gather(inv_rows.at[slot],
                                     [rows, jnp.full((16,), pos, jnp.int32)])
                b = plsc.load_gather(stk_rows.at[slot],
                                     [rows, jnp.full((16,), d, jnp.int32)])
                acc = acc + a * b
            out_v[j, pl.ds(t * 16, 16)] = acc
            return 0
        lax.fori_loop(0, CHUNK // 16, strip, 0)

    pltpu.sync_copy(out_v, out_hbm.at[pl.ds(base, NCHUNK)])


@jax.jit
def kernel(investors, stocks, W_investor, W_stock):
    mesh = plsc.VectorSubcoreMesh(core_axis_name="c", subcore_axis_name="s")
    k = functools.partial(
        pl.kernel,
        mesh=mesh,
        compiler_params=pltpu.CompilerParams(needs_layout_passes=False,
                                             use_tc_tiling_on_sc=False),
        out_type=jax.ShapeDtypeStruct((BATCH // CHUNK, CHUNK), jnp.float32),
        scratch_types=[
            pltpu.VMEM((NCHUNK, CHUNK), jnp.int32),
            pltpu.VMEM((NCHUNK, CHUNK), jnp.int32),
            pltpu.VMEM((2, CHUNK, LATENT), jnp.bfloat16),
            pltpu.VMEM((2, CHUNK, LATENT), jnp.float32),
            pltpu.VMEM((2, CHUNK, LATENT), jnp.float32),
            pltpu.VMEM((NCHUNK, CHUNK), jnp.float32),
            pltpu.SemaphoreType.DMA((2,)),
        ],
    )(_sc_kernel)
    out = k(investors.reshape(BATCH // CHUNK, CHUNK),
            stocks.reshape(BATCH // CHUNK, CHUNK),
            W_investor.astype(jnp.bfloat16), W_stock)
    return out.reshape(BATCH)


# speedup vs baseline: 3.6418x; 3.6418x over previous
import functools

import jax
import jax.numpy as jnp
from jax import lax
from jax.experimental import pallas as pl
from jax.experimental.pallas import tpu as pltpu
from jax.experimental.pallas import tpu_sc as plsc

LATENT = 32
BATCH = 16384
NW = 32
B_PER_W = BATCH // NW
CHUNK = 128
NCHUNK = B_PER_W // CHUNK
RING = 8
INV_N = 1000000
MAX_SLAB_START = ((INV_N - 128) // 128) * 128


def _slab_start(i):
    return pl.multiple_of((i >> 7) << 7, 128)


def _gather_kernel(inv_hbm, wti_hbm, rows_hbm, idx_v, slabs, ovec, idx_s,
                   sems):
    wid = lax.axis_index("s") * 2 + lax.axis_index("c")
    base = wid * B_PER_W

    pltpu.sync_copy(inv_hbm.at[pl.ds(base, B_PER_W)], idx_v)

    def spill(t, _):
        v = idx_v[pl.ds(t * 16, 16)]
        for l in range(16):
            idx_s[t * 16 + l] = v[l]
        return 0
    lax.fori_loop(0, B_PER_W // 16, spill, 0)

    def fire(e, slot):
        c = _slab_start(idx_s[e])
        return pltpu.async_copy(wti_hbm.at[:, pl.ds(c, 128)],
                                slabs.at[slot], sems.at[slot])

    for k in range(RING):
        fire(k, k)

    dlo = jnp.arange(16, dtype=jnp.int32)
    dhi = dlo + 16

    def body(e, _):
        slot = lax.rem(e, RING)
        pltpu.make_async_copy(wti_hbm.at[:, pl.ds(0, 128)],
                              slabs.at[slot], sems.at[slot]).wait()
        i = idx_s[e]
        col = jnp.broadcast_to(i - _slab_start(i), (16,))
        slotv = jnp.broadcast_to(slot, (16,))
        a_lo = plsc.load_gather(slabs, [slotv, dlo, col])
        a_hi = plsc.load_gather(slabs, [slotv, dhi, col])
        ovec[pl.ds(e * LATENT, 16)] = a_lo
        ovec[pl.ds(e * LATENT + 16, 16)] = a_hi

        @pl.when(e < B_PER_W - RING)
        def _():
            fire(e + RING, slot)
        return 0

    lax.fori_loop(0, B_PER_W, body, 0)
    pltpu.sync_copy(ovec, rows_hbm.at[pl.ds(base * LATENT, B_PER_W * LATENT)])


def _dot_kernel(stk_hbm, ws_hbm, rowsflat_hbm, out_hbm,
                stk_idx, inv_lin, stk_rows, out_v, sems):
    wid = lax.axis_index("s") * 2 + lax.axis_index("c")
    base = wid * NCHUNK

    pltpu.sync_copy(stk_hbm.at[pl.ds(base, NCHUNK)], stk_idx)
    pltpu.sync_copy(
        rowsflat_hbm.at[pl.ds(wid * B_PER_W * LATENT, B_PER_W * LATENT)],
        inv_lin)

    def fire(j, slot):
        return pltpu.async_copy(ws_hbm.at[stk_idx.at[j]],
                                stk_rows.at[slot], sems.at[slot])

    pending = fire(0, 0)
    lanes = jnp.arange(16, dtype=jnp.int32)
    for j in range(NCHUNK):
        slot = j % 2
        nxt = fire(j + 1, (j + 1) % 2) if j + 1 < NCHUNK else None
        pending.wait()
        pending = nxt

        def strip(t, _):
            rows = j * CHUNK + t * 16 + lanes
            acc = jnp.zeros((16,), jnp.float32)
            for d in range(LATENT):
                a = plsc.load_gather(inv_lin, [rows * LATENT + d])
                b = plsc.load_gather(stk_rows.at[slot],
                                     [rows - j * CHUNK,
                                      jnp.full((16,), d, jnp.int32)])
                acc = acc + a * b
            out_v[j, pl.ds(t * 16, 16)] = acc
            return 0
        lax.fori_loop(0, CHUNK // 16, strip, 0)

    pltpu.sync_copy(out_v, out_hbm.at[pl.ds(base, NCHUNK)])


@jax.jit
def kernel(investors, stocks, W_investor, W_stock):
    mesh = plsc.VectorSubcoreMesh(core_axis_name="c", subcore_axis_name="s")
    gather_k = functools.partial(
        pl.kernel,
        mesh=mesh,
        compiler_params=pltpu.CompilerParams(needs_layout_passes=False,
                                             disable_bounds_checks=True),
        out_type=jax.ShapeDtypeStruct((BATCH * LATENT,), jnp.float32),
        scratch_types=[
            pltpu.VMEM((B_PER_W,), jnp.int32),
            pltpu.VMEM((RING, LATENT, CHUNK), jnp.float32),
            pltpu.VMEM((B_PER_W * LATENT,), jnp.float32),
            pltpu.SMEM((B_PER_W,), jnp.int32),
            pltpu.SemaphoreType.DMA((RING,)),
        ],
    )(_gather_kernel)
    inv_rows_flat = gather_k(investors, W_investor.T)

    dot_k = functools.partial(
        pl.kernel,
        mesh=mesh,
        compiler_params=pltpu.CompilerParams(needs_layout_passes=False,
                                             use_tc_tiling_on_sc=False),
        out_type=jax.ShapeDtypeStruct((BATCH // CHUNK, CHUNK), jnp.float32),
        scratch_types=[
            pltpu.VMEM((NCHUNK, CHUNK), jnp.int32),
            pltpu.VMEM((B_PER_W * LATENT,), jnp.float32),
            pltpu.VMEM((2, CHUNK, LATENT), jnp.float32),
            pltpu.VMEM((NCHUNK, CHUNK), jnp.float32),
            pltpu.SemaphoreType.DMA((2,)),
        ],
    )(_dot_kernel)
    out = dot_k(stocks.reshape(BATCH // CHUNK, CHUNK), W_stock,
                inv_rows_flat)
    return out.reshape(BATCH)
